# Initial kernel scaffold; baseline (speedup 1.0000x reference)
import jax
import jax.numpy as jnp
from jax.experimental import pallas as pl


def kernel(x, edge_index, params):
    raise NotImplementedError("write your pallas kernel here")


# speedup vs baseline: 3.8082x; 3.8082x over previous
import functools

import jax
import jax.numpy as jnp
from jax import lax
from jax.experimental import pallas as pl
from jax.experimental.pallas import tpu as pltpu
from jax.experimental.pallas import tpu_sc as plsc

N, E, D, H, L = 10000, 320000, 128, 128, 4

NC, NS = 2, 16
NW = NC * NS
BLK = 128
KPB = 79
E_PAD = NW * KPB * BLK
ACC_ROWS = 10112
ROWS_PER_SUB = ACC_ROWS // NS

BLKN = 1000
GRID_N = N // BLKN


def _sc_segment_sum(h, src3, dst3):
  mesh = plsc.VectorSubcoreMesh(core_axis_name="c", subcore_axis_name="s")

  @functools.partial(
      pl.kernel,
      out_type=jax.ShapeDtypeStruct((NC, ACC_ROWS, D), jnp.float32),
      mesh=mesh,
      scratch_types=[
          pltpu.VMEM((KPB, BLK), jnp.int32),
          pltpu.VMEM((KPB, BLK), jnp.int32),
          pltpu.VMEM((BLK, D), jnp.float32),
          pltpu.VMEM_SHARED((ACC_ROWS, D), jnp.float32),
          pltpu.SemaphoreType.DMA,
      ],
  )
  def k(h_hbm, src_hbm, dst_hbm, out_hbm, src_v, dst_v, rows_v, acc, sem):
    c = lax.axis_index("c")
    s = lax.axis_index("s")
    wid = s * NC + c

    pltpu.sync_copy(src_hbm.at[wid], src_v)
    pltpu.sync_copy(dst_hbm.at[wid], dst_v)

    @pl.loop(0, BLK)
    def _(r):
      @pl.loop(0, D // 16)
      def _(j):
        rows_v[r, pl.ds(j * 16, 16)] = jnp.zeros((16,), jnp.float32)

    zbase = s * ROWS_PER_SUB

    @pl.loop(0, ROWS_PER_SUB // BLK)
    def _(t):
      pltpu.sync_copy(rows_v, acc.at[pl.ds(zbase + t * BLK, BLK)])

    rem = ROWS_PER_SUB % BLK
    if rem:
      pltpu.sync_copy(
          rows_v.at[pl.ds(0, rem)],
          acc.at[pl.ds(zbase + (ROWS_PER_SUB // BLK) * BLK, rem)])

    plsc.subcore_barrier()

    @pl.loop(0, KPB)
    def _(kb):
      pltpu.async_copy(h_hbm.at[src_v.at[kb]], rows_v, sem).wait()
      pltpu.sync_copy(rows_v, acc.at[dst_v.at[kb]], add=True)

    plsc.subcore_barrier()

    pltpu.sync_copy(acc.at[pl.ds(zbase, ROWS_PER_SUB)],
                    out_hbm.at[c, pl.ds(zbase, ROWS_PER_SUB)])

  return k(h, src3, dst3)


def _ln(t, g, b):
  mu = jnp.mean(t, axis=-1, keepdims=True)
  var = jnp.mean((t - mu) * (t - mu), axis=-1, keepdims=True)
  return (t - mu) * jax.lax.rsqrt(var + 1e-5) * g + b


def _tc_layer_body(h_ref, parts_ref, w1_ref, b1_ref, g1_ref, be1_ref,
                   w2_ref, b2_ref, g2_ref, be2_ref, eps_ref, bng_ref, bnb_ref,
                   out_ref, sum_ref, max_ref):
  i = pl.program_id(0)
  z = (1.0 + eps_ref[0, 0]) * h_ref[...] + parts_ref[0] + parts_ref[1]
  t = jnp.dot(z, w1_ref[...], preferred_element_type=jnp.float32, precision=lax.Precision.HIGHEST) + b1_ref[...]
  t = jnp.maximum(_ln(t, g1_ref[...], be1_ref[...]), 0.0)
  t = jnp.dot(t, w2_ref[...], preferred_element_type=jnp.float32, precision=lax.Precision.HIGHEST) + b2_ref[...]
  t = jnp.maximum(_ln(t, g2_ref[...], be2_ref[...]), 0.0)
  hout = jnp.maximum(_ln(t, bng_ref[...], bnb_ref[...]), 0.0)
  out_ref[...] = hout
  bsum = jnp.sum(hout, axis=0, keepdims=True)
  bmax = jnp.max(hout, axis=0, keepdims=True)

  @pl.when(i == 0)
  def _():
    sum_ref[...] = bsum
    max_ref[...] = bmax

  @pl.when(i != 0)
  def _():
    sum_ref[...] = sum_ref[...] + bsum
    max_ref[...] = jnp.maximum(max_ref[...], bmax)


def _tc_layer(h, parts, lp):
  row = lambda v: v.reshape(1, -1)
  vec_spec = pl.BlockSpec((1, H), lambda i: (0, 0))
  mat_spec = pl.BlockSpec((H, H), lambda i: (0, 0))
  return pl.pallas_call(
      _tc_layer_body,
      grid=(GRID_N,),
      in_specs=[
          pl.BlockSpec((BLKN, D), lambda i: (i, 0)),
          pl.BlockSpec((NC, BLKN, D), lambda i: (0, i, 0)),
          mat_spec, vec_spec, vec_spec, vec_spec,
          mat_spec, vec_spec, vec_spec, vec_spec,
          pl.BlockSpec((1, 1), lambda i: (0, 0)),
          vec_spec, vec_spec,
      ],
      out_specs=[
          pl.BlockSpec((BLKN, D), lambda i: (i, 0)),
          pl.BlockSpec((1, D), lambda i: (0, 0)),
          pl.BlockSpec((1, D), lambda i: (0, 0)),
      ],
      out_shape=[
          jax.ShapeDtypeStruct((N, D), jnp.float32),
          jax.ShapeDtypeStruct((1, D), jnp.float32),
          jax.ShapeDtypeStruct((1, D), jnp.float32),
      ],
  )(h, parts, lp["W1"], row(lp["b1"]), row(lp["g1"]), row(lp["be1"]),
    lp["W2"], row(lp["b2"]), row(lp["g2"]), row(lp["be2"]),
    lp["eps"].reshape(1, 1), row(lp["bng"]), row(lp["bnb"]))


def _head_body(xsum_ref, xmax_ref, wa_ref, ba_ref, ga_ref, bea_ref,
               wb_ref, bb_ref, gb_ref, beb_ref, wc_ref, bc_ref, out_ref):
  xsum = xsum_ref[...]
  xp = jnp.concatenate([xsum, xsum * (1.0 / N), xmax_ref[...]], axis=-1)
  t = jnp.dot(xp, wa_ref[...], preferred_element_type=jnp.float32, precision=lax.Precision.HIGHEST) + ba_ref[...]
  t = jnp.maximum(_ln(t, ga_ref[...], bea_ref[...]), 0.0)
  t = jnp.dot(t, wb_ref[...], preferred_element_type=jnp.float32, precision=lax.Precision.HIGHEST) + bb_ref[...]
  t = jnp.maximum(_ln(t, gb_ref[...], beb_ref[...]), 0.0)
  out_ref[...] = jnp.dot(t, wc_ref[...],
                         preferred_element_type=jnp.float32, precision=lax.Precision.HIGHEST) + bc_ref[...]


def _head(xsum, xmax, lin):
  row = lambda v: v.reshape(1, -1)
  return pl.pallas_call(
      _head_body,
      out_shape=jax.ShapeDtypeStruct((1, 1), jnp.float32),
  )(xsum, xmax, lin["Wa"], row(lin["ba"]), row(lin["ga"]), row(lin["bea"]),
    lin["Wb"], row(lin["bb"]), row(lin["gb"]), row(lin["beb"]),
    lin["Wc"], row(lin["bc"]))


def kernel(x, edge_index, params):
  pad = E_PAD - E
  src3 = jnp.concatenate(
      [edge_index[0], jnp.zeros((pad,), jnp.int32)]).reshape(NW, KPB, BLK)
  dst3 = jnp.concatenate(
      [edge_index[1], jnp.full((pad,), N, jnp.int32)]).reshape(NW, KPB, BLK)

  h = x
  sums, maxs = [], []
  for lp in params["layers"]:
    parts = _sc_segment_sum(h, src3, dst3)
    h, ls, lm = _tc_layer(h, parts, lp)
    sums.append(ls)
    maxs.append(lm)

  xsum = jnp.concatenate(sums, axis=-1)
  xmax = jnp.concatenate(maxs, axis=-1)
  out = _head(xsum, xmax, params["lin"])
  return out.reshape(-1)
